# baseline (speedup 1.0000x reference)
import dataclasses
import functools

import jax
import jax.numpy as jnp
from jax import lax
from jax.experimental import pallas as pl
from jax.experimental.pallas import tpu as pltpu
from jax.experimental.pallas import tpu_sc as plsc

_VOCAB = 1000
_BOS = 1


def _sc_pred_kernel(ids_hbm, pred_hbm, toks_ref, counts_ref, predbuf_ref,
                    dma_sem, *, num_cores, rows_per_worker, seqlen):
    wid = lax.axis_index("s") * num_cores + lax.axis_index("c")
    base = wid * rows_per_worker

    zeros16 = jnp.zeros((16,), jnp.int32)
    ones16 = jnp.ones((16,), jnp.int32)
    lane0 = lax.iota(jnp.int32, 16) == 0

    cp_in = pltpu.make_async_copy(
        ids_hbm.at[pl.ds(base, rows_per_worker)], toks_ref, dma_sem)
    cp_in.start()

    @pl.loop(0, 1024, step=16)
    def _zero(i):
        counts_ref[pl.ds(i, 16)] = zeros16

    cp_in.wait()

    tail_off = seqlen - 16
    offs = list(range(0, seqlen - 15, 16))
    lane_masks = [None] * len(offs)
    if offs[-1] + 16 < seqlen:
        offs.append(tail_off)
        lane_masks.append(lax.iota(jnp.int32, 16) >= 16 - (seqlen - offs[-2] - 16))
    for r in range(rows_per_worker):
        toks = [toks_ref[r, pl.ds(o, 16)] for o in offs]
        valids = [tok > 1 if m is None else (tok > 1) & m
                  for tok, m in zip(toks, lane_masks)]
        for tok, valid in zip(toks, valids):
            plsc.addupdate_scatter(counts_ref, [tok], ones16, mask=valid)
        kmax = zeros16
        for tok, valid in zip(toks, valids):
            cnt = plsc.load_gather(counts_ref, [tok])
            key = (cnt << 10) | (jnp.int32(1023) - tok)
            kmax = jnp.maximum(kmax, jnp.where(valid, key, 0))
            plsc.store_scatter(counts_ref, [tok], zeros16, mask=valid)
        k = jnp.max(kmax, axis=0)
        p = jnp.where(k >> 10 > 0, jnp.int32(1023) - (k & jnp.int32(1023)),
                      jnp.int32(_BOS))
        plsc.store_scatter(predbuf_ref, [jnp.full((16,), r, jnp.int32)],
                           jnp.full((16,), p, jnp.int32), mask=lane0)

    pltpu.sync_copy(predbuf_ref, pred_hbm.at[0, pl.ds(base, rows_per_worker)])


def _write_kernel(pred_ref, out_ref):
    vocab, bsz = out_ref.shape[1], out_ref.shape[2]
    pred = jnp.broadcast_to(pred_ref[...], (vocab, bsz))
    viota = jax.lax.broadcasted_iota(jnp.int32, (vocab, bsz), 0)
    out_ref[0] = jnp.where(viota == pred, jnp.float32(6.0), jnp.float32(-6.0))


def kernel(input_ids):
    bsz, seqlen = input_ids.shape

    cp = pltpu.CompilerParams()
    if "needs_layout_passes" in pltpu.CompilerParams.__dataclass_fields__:
        cp = dataclasses.replace(cp, needs_layout_passes=False)
    mesh = plsc.VectorSubcoreMesh(core_axis_name="c", subcore_axis_name="s")
    num_workers = mesh.num_cores * mesh.num_subcores
    rpw = bsz // num_workers
    sc_pred = pl.kernel(
        functools.partial(_sc_pred_kernel, num_cores=mesh.num_cores,
                          rows_per_worker=rpw, seqlen=seqlen),
        out_type=jax.ShapeDtypeStruct((1, bsz), jnp.int32),
        mesh=mesh,
        scratch_types=[
            pltpu.VMEM((rpw, seqlen), jnp.int32),
            pltpu.VMEM((1024,), jnp.int32),
            pltpu.VMEM((rpw,), jnp.int32),
            pltpu.SemaphoreType.DMA,
        ],
        compiler_params=cp,
    )
    pred = sc_pred(input_ids)

    out_t = pl.pallas_call(
        _write_kernel,
        grid=(seqlen,),
        in_specs=[pl.BlockSpec((1, bsz), lambda i: (0, 0))],
        out_specs=pl.BlockSpec((1, _VOCAB, bsz), lambda i: (i, 0, 0)),
        out_shape=jax.ShapeDtypeStruct((seqlen, _VOCAB, bsz), jnp.float32),
    )(pred)
    return jnp.transpose(out_t, (2, 0, 1))


# speedup vs baseline: 1.0070x; 1.0070x over previous
import functools

import jax
import jax.numpy as jnp
from jax import lax
from jax.experimental import pallas as pl
from jax.experimental.pallas import tpu as pltpu
from jax.experimental.pallas import tpu_sc as plsc

_VOCAB = 1000
_BOS = 1


def _sc_pred_kernel(ids_hbm, pred_hbm, toks_ref, counts_ref, predbuf_ref,
                    dma_sem, *, num_cores, rows_per_worker, seqlen):
    wid = lax.axis_index("s") * num_cores + lax.axis_index("c")
    base = wid * rows_per_worker

    zeros16 = jnp.zeros((16,), jnp.int32)
    ones16 = jnp.ones((16,), jnp.int32)
    lane0 = lax.iota(jnp.int32, 16) == 0

    cp_in = pltpu.make_async_copy(
        ids_hbm.at[pl.ds(base, rows_per_worker)], toks_ref, dma_sem)
    cp_in.start()

    @pl.loop(0, 1024, step=16)
    def _zero(i):
        counts_ref[pl.ds(i, 16)] = zeros16

    cp_in.wait()

    tail_off = seqlen - 16
    offs = list(range(0, seqlen - 15, 16))
    lane_masks = [None] * len(offs)
    if offs[-1] + 16 < seqlen:
        offs.append(tail_off)
        lane_masks.append(lax.iota(jnp.int32, 16) >= 16 - (seqlen - offs[-2] - 16))
    for r in range(rows_per_worker):
        toks = [toks_ref[r, pl.ds(o, 16)] for o in offs]
        valids = [tok > 1 if m is None else (tok > 1) & m
                  for tok, m in zip(toks, lane_masks)]
        for tok, valid in zip(toks, valids):
            plsc.addupdate_scatter(counts_ref, [tok], ones16, mask=valid)
        kmax = zeros16
        for tok, valid in zip(toks, valids):
            cnt = plsc.load_gather(counts_ref, [tok])
            key = (cnt << 10) | (jnp.int32(1023) - tok)
            kmax = jnp.maximum(kmax, jnp.where(valid, key, 0))
            plsc.store_scatter(counts_ref, [tok], zeros16, mask=valid)
        k = jnp.max(kmax, axis=0)
        p = jnp.where(k >> 10 > 0, jnp.int32(1023) - (k & jnp.int32(1023)),
                      jnp.int32(_BOS))
        plsc.store_scatter(predbuf_ref, [jnp.full((16,), r, jnp.int32)],
                           jnp.full((16,), p, jnp.int32), mask=lane0)

    pltpu.sync_copy(predbuf_ref, pred_hbm.at[0, pl.ds(base, rows_per_worker)])


def _write_kernel(pred_ref, out_ref):
    vocab, bsz = out_ref.shape[1], out_ref.shape[2]
    pred = jnp.broadcast_to(pred_ref[...], (vocab, bsz))
    viota = jax.lax.broadcasted_iota(jnp.int32, (vocab, bsz), 0)
    out_ref[0] = jnp.where(viota == pred, jnp.float32(6.0), jnp.float32(-6.0))


def kernel(input_ids):
    bsz, seqlen = input_ids.shape

    cp = pltpu.CompilerParams(needs_layout_passes=False)
    mesh = plsc.VectorSubcoreMesh(core_axis_name="c", subcore_axis_name="s")
    num_workers = mesh.num_cores * mesh.num_subcores
    rpw = bsz // num_workers
    sc_pred = pl.kernel(
        functools.partial(_sc_pred_kernel, num_cores=mesh.num_cores,
                          rows_per_worker=rpw, seqlen=seqlen),
        out_type=jax.ShapeDtypeStruct((1, bsz), jnp.int32),
        mesh=mesh,
        scratch_types=[
            pltpu.VMEM((rpw, seqlen), jnp.int32),
            pltpu.VMEM((1024,), jnp.int32),
            pltpu.VMEM((rpw,), jnp.int32),
            pltpu.SemaphoreType.DMA,
        ],
        compiler_params=cp,
    )
    pred = sc_pred(input_ids)

    out_t = pl.pallas_call(
        _write_kernel,
        grid=(seqlen,),
        in_specs=[pl.BlockSpec((1, bsz), lambda i: (0, 0))],
        out_specs=pl.BlockSpec((1, _VOCAB, bsz), lambda i: (i, 0, 0)),
        out_shape=jax.ShapeDtypeStruct((seqlen, _VOCAB, bsz), jnp.float32),
    )(pred)
    return jnp.transpose(out_t, (2, 0, 1))
